# baseline (speedup 1.0000x reference)
import functools

import jax
import jax.numpy as jnp
from jax import lax
from jax.experimental import pallas as pl
from jax.experimental.pallas import tpu as pltpu
from jax.experimental.pallas import tpu_sc as plsc

N_NODES = 50000
NPAD = 50176
D = 64
DH = D // 2
E = 800000
E_PAD = 819200
B = 4096
NC = 2
NS = 16
L = 16

SUB = 128
BLK = 8
EPT = E_PAD // NS
NBLOCK = EPT // (SUB * BLK)
IDX_ROWS_PT = EPT // SUB

ACC_PT = NPAD // NS
STAGE = 224

_mesh = plsc.VectorSubcoreMesh(
    core_axis_name="c", subcore_axis_name="s", num_cores=NC, num_subcores=NS)


NG = 4
CPI = 2 * BLK


BPT = B // (NC * NS)


@functools.partial(
    pl.kernel,
    out_type=[jax.ShapeDtypeStruct((NC, NPAD, DH), jnp.float32)
              for _ in range(3)],
    mesh=_mesh,
    scratch_types=[
        pltpu.VMEM((BLK, SUB), jnp.int32),
        pltpu.VMEM((BLK, SUB), jnp.int32),
        pltpu.VMEM((BLK, SUB), jnp.float32),
        pltpu.VMEM((BLK, SUB), jnp.int32),
        pltpu.VMEM((BLK, SUB), jnp.int32),
        pltpu.VMEM((BLK, SUB), jnp.float32),
        pltpu.VMEM((SUB, DH), jnp.float32),
        pltpu.VMEM((SUB, DH), jnp.float32),
        pltpu.VMEM((SUB, DH), jnp.float32),
        pltpu.VMEM((SUB, DH), jnp.float32),
        pltpu.VMEM((STAGE, DH), jnp.float32),
        pltpu.VMEM_SHARED((NPAD, DH), jnp.float32),
        pltpu.SemaphoreType.DMA,
        pltpu.SemaphoreType.DMA,
        pltpu.SemaphoreType.DMA,
        pltpu.SemaphoreType.DMA,
        pltpu.SemaphoreType.DMA,
        pltpu.SemaphoreType.DMA,
        pltpu.SemaphoreType.DMA,
        pltpu.SemaphoreType.DMA,
        pltpu.SemaphoreType.DMA,
        pltpu.SemaphoreType.DMA,
        pltpu.SemaphoreType.DMA,
        pltpu.SemaphoreType.DMA,
        pltpu.SemaphoreType.DMA,
        pltpu.SemaphoreType.DMA,
        pltpu.SemaphoreType.DMA,
    ],
    compiler_params=pltpu.CompilerParams(use_tc_tiling_on_sc=False),
)
def _propagate(x0_h, src_h, dst_h, w_h, o1, o2, o3,
               srcA, dstA, wA, srcB, dstB, wB,
               rows0, rows1, rows2, rows3, stage, acc,
               pA0, pA1, pA2, pB0, pB1, pB2,
               g0, g1, g2, g3, s0, s1, s2, s3, sem_co):
    c = lax.axis_index("c")
    s = lax.axis_index("s")
    base = s * ACC_PT

    def _zero_row(r, _):
        for j in range(DH // L):
            stage[r, pl.ds(j * L, L)] = jnp.zeros((L,), jnp.float32)
        return 0
    lax.fori_loop(0, STAGE, _zero_row, 0)

    rows_bufs = (rows0, rows1, rows2, rows3)
    gsems = (g0, g1, g2, g3)
    ssems = (s0, s1, s2, s3)
    bufs = ((srcA, dstA, wA, (pA0, pA1, pA2)),
            (srcB, dstB, wB, (pB0, pB1, pB2)))

    def _prefetch(blk_idx, which):
        sv, dv, wv, sems = bufs[which]
        rb = s * IDX_ROWS_PT + blk_idx * BLK
        pltpu.async_copy(src_h.at[pl.ds(rb, BLK)], sv, sems[0])
        pltpu.async_copy(dst_h.at[pl.ds(rb, BLK)], dv, sems[1])
        pltpu.async_copy(w_h.at[pl.ds(rb, BLK)], wv, sems[2])

    def _wait_idx(which):
        sv, dv, wv, sems = bufs[which]
        pltpu.make_async_copy(src_h.at[pl.ds(0, BLK)], sv, sems[0]).wait()
        pltpu.make_async_copy(dst_h.at[pl.ds(0, BLK)], dv, sems[1]).wait()
        pltpu.make_async_copy(w_h.at[pl.ds(0, BLK)], wv, sems[2]).wait()

    def _wait_sca(b):
        pltpu.make_async_copy(rows_bufs[b], acc.at[dstA.at[0]],
                              ssems[b]).wait()

    def _one_layer(x_in, x_out):
        zcps = [pltpu.async_copy(
            stage, acc.at[pl.ds(base + p * STAGE, STAGE)], sem_co)
            for p in range(ACC_PT // STAGE)]
        for cp in zcps:
            cp.wait()
        plsc.subcore_barrier()

        def _gather(j, b):
            sv = bufs[j // BLK][0]
            return pltpu.async_copy(x_in.at[c].at[sv.at[j % BLK]],
                                    rows_bufs[b], gsems[b])

        _prefetch(0, 0)
        _prefetch(1, 1)

        def _iter(bi2, _):
            p0 = 2 * bi2
            _wait_idx(0)
            for j in range(3):
                _gather(j, j)
            for j in range(CPI):
                b = j % NG
                if j == 5:
                    _wait_idx(1)
                wX = bufs[j // BLK][2]
                dX = bufs[j // BLK][1]
                pltpu.make_async_copy(x_in.at[c].at[srcA.at[0]],
                                      rows_bufs[b], gsems[b]).wait()
                rq = rows_bufs[b]

                def _mul(m, _):
                    w16 = wX[j % BLK, pl.ds(m * L, L)]
                    for e in range(L):
                        ei = m * L + e
                        wb = jnp.broadcast_to(w16[e], (L,))
                        for jj in range(DH // L):
                            sl = pl.ds(jj * L, L)
                            rq[ei, sl] = rq[ei, sl] * wb
                    return 0
                lax.fori_loop(0, SUB // L, _mul, 0)

                pltpu.async_copy(rq, acc.at[dX.at[j % BLK]], ssems[b],
                                 add=True)
                jn = j + 3
                if jn < CPI:
                    bn = jn % NG
                    if jn >= NG:
                        _wait_sca(bn)
                    _gather(jn, bn)
                if j == BLK + 1:
                    _prefetch(jnp.minimum(p0 + 2, NBLOCK - 1), 0)
            for b in range(NG):
                _wait_sca(b)
            _prefetch(jnp.minimum(p0 + 3, NBLOCK - 1), 1)
            return 0
        lax.fori_loop(0, NBLOCK // 2, _iter, 0)
        _wait_idx(0)
        _wait_idx(1)
        plsc.subcore_barrier()

        cps = [pltpu.async_copy(
            acc.at[pl.ds(base + p * STAGE, STAGE)],
            x_out.at[c].at[pl.ds(base + p * STAGE, STAGE)], sem_co)
            for p in range(ACC_PT // STAGE)]
        for cp in cps:
            cp.wait()

    _one_layer(x0_h, o1)
    _one_layer(o1, o2)
    _one_layer(o2, o3)


BPT = B // (NC * NS)


@functools.partial(
    pl.kernel,
    out_type=[jax.ShapeDtypeStruct((NC, B, DH), jnp.float32) for _ in range(6)],
    mesh=_mesh,
    scratch_types=[
        pltpu.VMEM((BPT,), jnp.int32),
        pltpu.VMEM((BPT, DH), jnp.float32),
        pltpu.VMEM((BPT, DH), jnp.float32),
        pltpu.SemaphoreType.DMA,
    ],
    compiler_params=pltpu.CompilerParams(use_tc_tiling_on_sc=False),
)
def _batch_gather(x0, x1, x2, x3, users_h, pos_h, neg_h,
                  su, u0, sp, p0, sn, n0,
                  idxv, bufa, bufb, sem):
    c = lax.axis_index("c")
    s = lax.axis_index("s")
    wid = s * NC + c
    base = wid * BPT

    for idx_h, item_side, osum, o0 in (
            (users_h, False, su, u0),
            (pos_h, True, sp, p0),
            (neg_h, True, sn, n0)):
        pltpu.sync_copy(idx_h.at[wid], idxv)

        def _adj(g, _):
            sl = pl.ds(g * L, L)
            idxv[sl] = idxv[sl] + N_NODES // 2
            return 0
        if item_side:
            lax.fori_loop(0, BPT // L, _adj, 0)

        for half in range(NC):
            pltpu.async_copy(x0.at[half].at[idxv], bufa, sem).wait()
            pltpu.sync_copy(bufa, o0.at[half].at[pl.ds(base, BPT)])
            for xk in (x1, x2, x3):
                pltpu.async_copy(xk.at[half].at[idxv], bufb, sem).wait()

                def _add(r, _):
                    for jj in range(DH // L):
                        sl = pl.ds(jj * L, L)
                        bufa[r, sl] = bufa[r, sl] + bufb[r, sl]
                    return 0
                lax.fori_loop(0, BPT, _add, 0)
            pltpu.sync_copy(bufa, osum.at[half].at[pl.ds(base, BPT)])


def _loss_body(su, sp, sn, u0, p0, n0, out):
    inv_k2 = 1.0 / 16.0
    pos_scores = jnp.sum(su[...] * sp[...], axis=(0, 2)) * inv_k2
    neg_scores = jnp.sum(su[...] * sn[...], axis=(0, 2)) * inv_k2
    loss = jnp.mean(jax.nn.softplus(neg_scores - pos_scores))
    reg = 0.5 * (jnp.sum(u0[...] ** 2) + jnp.sum(p0[...] ** 2)
                 + jnp.sum(n0[...] ** 2)) / float(B)
    out[...] = jnp.stack([loss, reg]).reshape(1, 2)


_loss_tc = pl.pallas_call(
    _loss_body,
    out_shape=jax.ShapeDtypeStruct((1, 2), jnp.float32),
)


def kernel(users_emb, items_emb, edge_weight, edge_index, users, pos, neg):
    zpad = jnp.zeros((NPAD - N_NODES, DH), jnp.float32)
    x0 = jnp.stack([
        jnp.concatenate([users_emb[:, :DH], items_emb[:, :DH], zpad], axis=0),
        jnp.concatenate([users_emb[:, DH:], items_emb[:, DH:], zpad], axis=0),
    ])

    epad_i = jnp.zeros((E_PAD - E,), jnp.int32)
    src2d = jnp.concatenate([edge_index[0], epad_i]).reshape(-1, SUB)
    dst2d = jnp.concatenate([edge_index[1], epad_i]).reshape(-1, SUB)
    w2d = jnp.concatenate(
        [edge_weight, jnp.zeros((E_PAD - E,), jnp.float32)]).reshape(-1, SUB)

    x1, x2, x3 = _propagate(x0, src2d, dst2d, w2d)

    su, u0, sp, p0, sn, n0 = _batch_gather(
        x0, x1, x2, x3,
        users.reshape(-1, BPT), pos.reshape(-1, BPT), neg.reshape(-1, BPT))
    out = _loss_tc(su, sp, sn, u0, p0, n0)
    return out.reshape(2)


# speedup vs baseline: 1.0070x; 1.0070x over previous
import functools

import jax
import jax.numpy as jnp
from jax import lax
from jax.experimental import pallas as pl
from jax.experimental.pallas import tpu as pltpu
from jax.experimental.pallas import tpu_sc as plsc

N_NODES = 50000
NPAD = 50176
D = 64
DH = D // 2
E = 800000
E_PAD = 819200
B = 4096
NC = 2
NS = 16
L = 16

SUB = 128
BLK = 8
EPT = E_PAD // NS
NBLOCK = EPT // (SUB * BLK)
IDX_ROWS_PT = EPT // SUB

ACC_PT = NPAD // NS
STAGE = 224

_mesh = plsc.VectorSubcoreMesh(
    core_axis_name="c", subcore_axis_name="s", num_cores=NC, num_subcores=NS)


NG = 4
CPI = 2 * BLK


BQ = B // NS


@functools.partial(
    pl.kernel,
    out_type=(
        [jax.ShapeDtypeStruct((NC, NPAD, DH), jnp.float32)
         for _ in range(3)] +
        [jax.ShapeDtypeStruct((NC, B, DH), jnp.float32) for _ in range(6)]),
    mesh=_mesh,
    scratch_types=[
        pltpu.VMEM((BLK, SUB), jnp.int32),
        pltpu.VMEM((BLK, SUB), jnp.int32),
        pltpu.VMEM((BLK, SUB), jnp.float32),
        pltpu.VMEM((BLK, SUB), jnp.int32),
        pltpu.VMEM((BLK, SUB), jnp.int32),
        pltpu.VMEM((BLK, SUB), jnp.float32),
        pltpu.VMEM((SUB, DH), jnp.float32),
        pltpu.VMEM((SUB, DH), jnp.float32),
        pltpu.VMEM((SUB, DH), jnp.float32),
        pltpu.VMEM((SUB, DH), jnp.float32),
        pltpu.VMEM((STAGE, DH), jnp.float32),
        pltpu.VMEM_SHARED((NPAD, DH), jnp.float32),
        pltpu.SemaphoreType.DMA,
        pltpu.SemaphoreType.DMA,
        pltpu.SemaphoreType.DMA,
        pltpu.SemaphoreType.DMA,
        pltpu.SemaphoreType.DMA,
        pltpu.SemaphoreType.DMA,
        pltpu.SemaphoreType.DMA,
        pltpu.SemaphoreType.DMA,
        pltpu.SemaphoreType.DMA,
        pltpu.SemaphoreType.DMA,
        pltpu.SemaphoreType.DMA,
        pltpu.SemaphoreType.DMA,
        pltpu.SemaphoreType.DMA,
        pltpu.SemaphoreType.DMA,
        pltpu.SemaphoreType.DMA,
    ],
    compiler_params=pltpu.CompilerParams(use_tc_tiling_on_sc=False),
)
def _propagate(x0_h, src_h, dst_h, w_h, users_h, pos_h, neg_h,
               o1, o2, o3, su, u0, sp, p0, sn, n0,
               srcA, dstA, wA, srcB, dstB, wB,
               rows0, rows1, rows2, rows3, stage, acc,
               pA0, pA1, pA2, pB0, pB1, pB2,
               g0, g1, g2, g3, s0, s1, s2, s3, sem_co):
    c = lax.axis_index("c")
    s = lax.axis_index("s")
    base = s * ACC_PT

    def _zero_row(r, _):
        for j in range(DH // L):
            stage[r, pl.ds(j * L, L)] = jnp.zeros((L,), jnp.float32)
        return 0
    lax.fori_loop(0, STAGE, _zero_row, 0)

    rows_bufs = (rows0, rows1, rows2, rows3)
    gsems = (g0, g1, g2, g3)
    ssems = (s0, s1, s2, s3)
    bufs = ((srcA, dstA, wA, (pA0, pA1, pA2)),
            (srcB, dstB, wB, (pB0, pB1, pB2)))

    def _prefetch(blk_idx, which):
        sv, dv, wv, sems = bufs[which]
        rb = s * IDX_ROWS_PT + blk_idx * BLK
        pltpu.async_copy(src_h.at[pl.ds(rb, BLK)], sv, sems[0])
        pltpu.async_copy(dst_h.at[pl.ds(rb, BLK)], dv, sems[1])
        pltpu.async_copy(w_h.at[pl.ds(rb, BLK)], wv, sems[2])

    def _wait_idx(which):
        sv, dv, wv, sems = bufs[which]
        pltpu.make_async_copy(src_h.at[pl.ds(0, BLK)], sv, sems[0]).wait()
        pltpu.make_async_copy(dst_h.at[pl.ds(0, BLK)], dv, sems[1]).wait()
        pltpu.make_async_copy(w_h.at[pl.ds(0, BLK)], wv, sems[2]).wait()

    def _wait_sca(b):
        pltpu.make_async_copy(rows_bufs[b], acc.at[dstA.at[0]],
                              ssems[b]).wait()

    def _one_layer(x_in, x_out):
        zcps = [pltpu.async_copy(
            stage, acc.at[pl.ds(base + p * STAGE, STAGE)], sem_co)
            for p in range(ACC_PT // STAGE)]
        for cp in zcps:
            cp.wait()
        plsc.subcore_barrier()

        def _gather(j, b):
            sv = bufs[j // BLK][0]
            return pltpu.async_copy(x_in.at[c].at[sv.at[j % BLK]],
                                    rows_bufs[b], gsems[b])

        _prefetch(0, 0)
        _prefetch(1, 1)

        def _iter(bi2, _):
            p0 = 2 * bi2
            _wait_idx(0)
            for j in range(3):
                _gather(j, j)
            for j in range(CPI):
                b = j % NG
                if j == 5:
                    _wait_idx(1)
                wX = bufs[j // BLK][2]
                dX = bufs[j // BLK][1]
                pltpu.make_async_copy(x_in.at[c].at[srcA.at[0]],
                                      rows_bufs[b], gsems[b]).wait()
                rq = rows_bufs[b]

                def _mul(m, _):
                    w16 = wX[j % BLK, pl.ds(m * L, L)]
                    for e in range(L):
                        ei = m * L + e
                        wb = jnp.broadcast_to(w16[e], (L,))
                        for jj in range(DH // L):
                            sl = pl.ds(jj * L, L)
                            rq[ei, sl] = rq[ei, sl] * wb
                    return 0
                lax.fori_loop(0, SUB // L, _mul, 0)

                pltpu.async_copy(rq, acc.at[dX.at[j % BLK]], ssems[b],
                                 add=True)
                jn = j + 3
                if jn < CPI:
                    bn = jn % NG
                    if jn >= NG:
                        _wait_sca(bn)
                    _gather(jn, bn)
                if j == BLK + 1:
                    _prefetch(jnp.minimum(p0 + 2, NBLOCK - 1), 0)
            for b in range(NG):
                _wait_sca(b)
            _prefetch(jnp.minimum(p0 + 3, NBLOCK - 1), 1)
            return 0
        lax.fori_loop(0, NBLOCK // 2, _iter, 0)
        _wait_idx(0)
        _wait_idx(1)
        plsc.subcore_barrier()

        cps = [pltpu.async_copy(
            acc.at[pl.ds(base + p * STAGE, STAGE)],
            x_out.at[c].at[pl.ds(base + p * STAGE, STAGE)], sem_co)
            for p in range(ACC_PT // STAGE)]
        for cp in cps:
            cp.wait()

    _one_layer(x0_h, o1)
    _one_layer(o1, o2)
    _one_layer(o2, o3)

    plsc.subcore_barrier()

    for idx_h, item_side, osum, o0 in (
            (users_h, False, su, u0),
            (pos_h, True, sp, p0),
            (neg_h, True, sn, n0)):
        pltpu.sync_copy(idx_h.at[s], srcA.at[pl.ds(0, 2)])
        if item_side:
            for r in range(2):
                for g in range(SUB // L):
                    sl = pl.ds(g * L, L)
                    srcA[r, sl] = srcA[r, sl] + N_NODES // 2
        for sub in range(2):
            rb = s * BQ + sub * SUB
            pltpu.async_copy(x0_h.at[c].at[srcA.at[sub]], rows0, g0).wait()
            pltpu.sync_copy(rows0, o0.at[c].at[pl.ds(rb, SUB)])
            for xk in (o1, o2, o3):
                pltpu.async_copy(xk.at[c].at[srcA.at[sub]], rows1, g1).wait()

                def _addrow(r, _):
                    for jj in range(DH // L):
                        sl = pl.ds(jj * L, L)
                        rows0[r, sl] = rows0[r, sl] + rows1[r, sl]
                    return 0
                lax.fori_loop(0, SUB, _addrow, 0)
            pltpu.sync_copy(rows0, osum.at[c].at[pl.ds(rb, SUB)])


def _loss_body(su, sp, sn, u0, p0, n0, out):
    inv_k2 = 1.0 / 16.0
    pos_scores = jnp.sum(su[...] * sp[...], axis=(0, 2)) * inv_k2
    neg_scores = jnp.sum(su[...] * sn[...], axis=(0, 2)) * inv_k2
    loss = jnp.mean(jax.nn.softplus(neg_scores - pos_scores))
    reg = 0.5 * (jnp.sum(u0[...] ** 2) + jnp.sum(p0[...] ** 2)
                 + jnp.sum(n0[...] ** 2)) / float(B)
    out[...] = jnp.stack([loss, reg]).reshape(1, 2)


_loss_tc = pl.pallas_call(
    _loss_body,
    out_shape=jax.ShapeDtypeStruct((1, 2), jnp.float32),
)


def kernel(users_emb, items_emb, edge_weight, edge_index, users, pos, neg):
    zpad = jnp.zeros((NPAD - N_NODES, DH), jnp.float32)
    x0 = jnp.stack([
        jnp.concatenate([users_emb[:, :DH], items_emb[:, :DH], zpad], axis=0),
        jnp.concatenate([users_emb[:, DH:], items_emb[:, DH:], zpad], axis=0),
    ])

    epad_i = jnp.zeros((E_PAD - E,), jnp.int32)
    src2d = jnp.concatenate([edge_index[0], epad_i]).reshape(-1, SUB)
    dst2d = jnp.concatenate([edge_index[1], epad_i]).reshape(-1, SUB)
    w2d = jnp.concatenate(
        [edge_weight, jnp.zeros((E_PAD - E,), jnp.float32)]).reshape(-1, SUB)

    _, _, _, su, u0, sp, p0, sn, n0 = _propagate(
        x0, src2d, dst2d, w2d,
        users.reshape(NS, 2, SUB), pos.reshape(NS, 2, SUB),
        neg.reshape(NS, 2, SUB))
    out = _loss_tc(su, sp, sn, u0, p0, n0)
    return out.reshape(2)
